# baseline (speedup 1.0000x reference)
import functools

import jax
import jax.numpy as jnp
from jax import lax
from jax.experimental import pallas as pl
from jax.experimental.pallas import tpu as pltpu
from jax.experimental.pallas import tpu_sc as plsc

B = 16384
F = 26
V = 100000
D = 16
ND = 13

NC = 2
NS = 16
NW = NC * NS
RPW = B // NW
CH = 64
NCHUNK = RPW // CH
TPC = CH * F // 128
ROWS_W = RPW * F // 128


def _fm_body(idxA_h, idxB_h, xdT_h, wb_h, e1_h, e2_h, out_h,
             idxA_v, idxB_v, e1_v, e2_v, xd_v, wb_v, redsum_v, o_v,
             sem1, sem2a, sem2b):
    wid = lax.axis_index("s") * NC + lax.axis_index("c")
    base = wid * RPW

    pltpu.sync_copy(idxA_h.at[pl.ds(wid * ROWS_W, ROWS_W)], idxA_v)
    pltpu.sync_copy(idxB_h.at[pl.ds(wid * ROWS_W, ROWS_W)], idxB_v)
    pltpu.sync_copy(xdT_h.at[:, pl.ds(base, RPW)], xd_v)
    pltpu.sync_copy(wb_h, wb_v)

    def fire1(j, carry):
        pltpu.async_copy(e1_h.at[idxB_v.at[j]], e1_v.at[j], sem1)
        return carry
    lax.fori_loop(0, ROWS_W, fire1, 0)

    lane = jax.lax.iota(jnp.int32, 16)

    def fire2(ci, slot, sem):
        for t in range(TPC):
            j = ci * TPC + t
            pltpu.async_copy(
                e2_h.at[idxA_v.at[j]],
                e2_v.at[slot, pl.ds(t * 128, 128)], sem)

    def drain2(slot, sem):
        for t in range(TPC):
            pltpu.make_async_copy(
                e2_h.at[idxA_v.at[0]],
                e2_v.at[slot, pl.ds(t * 128, 128)], sem).wait()

    fire2(0, 0, sem2a)

    def chunkbody(ci, carry):
        slot = lax.rem(ci, 2)

        @pl.when(ci + 1 < NCHUNK)
        def _fire_next():
            @pl.when(slot == 0)
            def _():
                fire2(ci + 1, 1, sem2b)

            @pl.when(slot == 1)
            def _():
                fire2(ci + 1, 0, sem2a)

        @pl.when(slot == 0)
        def _():
            drain2(0, sem2a)

        @pl.when(slot == 1)
        def _():
            drain2(1, sem2b)

        def groupbody(g, carry2):
            svec = jnp.zeros((16,), jnp.float32)
            for r16 in range(16):
                rb = (g * 16 + r16) * F
                acc = e2_v[slot, rb, :]
                acc2 = acc * acc
                for f in range(1, F):
                    v = e2_v[slot, rb + f, :]
                    acc = acc + v
                    acc2 = acc2 + v * v
                red = 0.5 * (acc * acc - acc2)
                parts = [red[d] for d in range(D)]
                while len(parts) > 1:
                    parts = [parts[i] + parts[i + 1]
                             for i in range(0, len(parts), 2)]
                svec = jnp.where(lane == r16, parts[0], svec)
            redsum_v[pl.ds(ci * CH + g * 16, 16)] = svec
            return carry2
        lax.fori_loop(0, CH // 16, groupbody, 0)
        return carry
    lax.fori_loop(0, NCHUNK, chunkbody, 0)

    def drain1(j, carry):
        pltpu.make_async_copy(e1_h.at[idxB_v.at[0]], e1_v.at[0], sem1).wait()
        return carry
    lax.fori_loop(0, ROWS_W, drain1, 0)

    wvec = wb_v[...]
    ws = [wvec[k] for k in range(ND)]
    b0 = wvec[15]

    def gbody(g, carry):
        gm = g // 8
        gc = (g % 8) * 16
        v = redsum_v[pl.ds(g * 16, 16)]
        for f in range(F):
            v = v + e1_v[f * (RPW // 128) + gm, pl.ds(gc, 16)]
        for k in range(ND):
            v = v + xd_v[k, pl.ds(g * 16, 16)] * ws[k]
        v = v + b0
        o_v[pl.ds(g * 16, 16)] = 1.0 / (1.0 + jnp.exp(-v))
        return carry
    lax.fori_loop(0, RPW // 16, gbody, 0)

    pltpu.sync_copy(o_v, out_h.at[pl.ds(base, RPW)])


@functools.partial(
    pl.kernel,
    mesh=plsc.VectorSubcoreMesh(core_axis_name="c", subcore_axis_name="s"),
    out_type=jax.ShapeDtypeStruct((B,), jnp.float32),
    compiler_params=pltpu.CompilerParams(use_tc_tiling_on_sc=False),
    scratch_types=[
        pltpu.VMEM((NW * ROWS_W // NW, 128), jnp.int32),
        pltpu.VMEM((ROWS_W, 128), jnp.int32),
        pltpu.VMEM((ROWS_W, 128), jnp.float32),
        pltpu.VMEM((2, CH * F, D), jnp.float32),
        pltpu.VMEM((ND, RPW), jnp.float32),
        pltpu.VMEM((16,), jnp.float32),
        pltpu.VMEM((RPW,), jnp.float32),
        pltpu.VMEM((RPW,), jnp.float32),
        pltpu.SemaphoreType.DMA,
        pltpu.SemaphoreType.DMA,
        pltpu.SemaphoreType.DMA,
    ],
)
def _fm_kernel(idxA_h, idxB_h, xdT_h, wb_h, e1_h, e2_h, out_h, *scratch):
    _fm_body(idxA_h, idxB_h, xdT_h, wb_h, e1_h, e2_h, out_h, *scratch)


def kernel(X_sparse, X_dense, emb1_tables, emb2_tables, W_dense, b_dense):
    off = (jnp.arange(F, dtype=jnp.int32) * V)[None, :]
    flat = X_sparse + off
    idxA = flat.reshape(NW * ROWS_W, 128)
    idxB = (flat.T.reshape(F, NW, RPW // 128, 128)
            .transpose(1, 0, 2, 3).reshape(NW * ROWS_W, 128))
    xdT = X_dense.T
    wb = jnp.concatenate([
        W_dense[:, 0],
        jnp.zeros((2,), jnp.float32),
        b_dense,
    ])
    e1 = emb1_tables.reshape(F * V)
    e2 = emb2_tables.reshape(F * V, D)
    out = _fm_kernel(idxA, idxB, xdT, wb, e1, e2)
    return out.reshape(B, 1)


# speedup vs baseline: 1.0011x; 1.0011x over previous
import functools

import jax
import jax.numpy as jnp
from jax import lax
from jax.experimental import pallas as pl
from jax.experimental.pallas import tpu as pltpu
from jax.experimental.pallas import tpu_sc as plsc

B = 16384
F = 26
V = 100000
D = 16
ND = 13

NC = 2
NS = 16
NW = NC * NS
RPW = B // NW
CH = 64
NCHUNK = RPW // CH
TPC = CH * F // 128
ROWS_W = RPW * F // 128
E1_PAD = ROWS_W * 128 + 16
XD_PAD = RPW * ND + 16


def _fm_body(idx_h, xd_h, wb_h, e1_h, e2_h, out_h,
             idx_v, e1_v, e2_v, xd_v, wb_v, redsum_v, o_v,
             sem1, sem2a, sem2b):
    wid = lax.axis_index("s") * NC + lax.axis_index("c")
    base = wid * RPW

    pltpu.sync_copy(idx_h.at[pl.ds(wid * ROWS_W, ROWS_W)], idx_v)
    pltpu.sync_copy(xd_h.at[pl.ds(base * ND, RPW * ND)],
                    xd_v.at[pl.ds(0, RPW * ND)])
    pltpu.sync_copy(wb_h, wb_v)

    zero16 = jnp.zeros((16,), jnp.float32)
    e1_v[pl.ds(ROWS_W * 128, 16)] = zero16
    xd_v[pl.ds(RPW * ND, 16)] = zero16

    def fire2(ci, slot, sem):
        for t in range(TPC):
            j = ci * TPC + t
            pltpu.async_copy(
                e2_h.at[idx_v.at[j]],
                e2_v.at[slot, pl.ds(t * 128, 128)], sem)

    def drain2(slot, sem):
        for t in range(TPC):
            pltpu.make_async_copy(
                e2_h.at[idx_v.at[0]],
                e2_v.at[slot, pl.ds(t * 128, 128)], sem).wait()

    fire2(0, 0, sem2a)

    def fire1(j, carry):
        pltpu.async_copy(e1_h.at[idx_v.at[j]],
                         e1_v.at[pl.ds(j * 128, 128)], sem1)
        return carry
    lax.fori_loop(0, ROWS_W, fire1, 0)

    def drain1(j, carry):
        pltpu.make_async_copy(e1_h.at[idx_v.at[0]],
                              e1_v.at[pl.ds(0, 128)], sem1).wait()
        return carry
    lax.fori_loop(0, ROWS_W, drain1, 0)

    lane = jax.lax.iota(jnp.int32, 16)
    m10 = jnp.where(lane < F - 16, 1.0, 0.0).astype(jnp.float32)
    wvec = wb_v[...]
    wpad = jnp.where(lane < ND, wvec, 0.0).astype(jnp.float32)
    b0 = wvec[15]

    def chunkbody(ci, carry):
        slot = lax.rem(ci, 2)

        @pl.when(ci + 1 < NCHUNK)
        def _fire_next():
            @pl.when(slot == 0)
            def _():
                fire2(ci + 1, 1, sem2b)

            @pl.when(slot == 1)
            def _():
                fire2(ci + 1, 0, sem2a)

        @pl.when(slot == 0)
        def _():
            drain2(0, sem2a)

        @pl.when(slot == 1)
        def _():
            drain2(1, sem2b)

        def groupbody(g, carry2):
            svec = jnp.zeros((16,), jnp.float32)
            for r16 in range(16):
                r = g * 16 + r16
                rb = r * F
                acc = e2_v[slot, rb, :]
                acc2 = acc * acc
                for f in range(1, F):
                    v = e2_v[slot, rb + f, :]
                    acc = acc + v
                    acc2 = acc2 + v * v
                tot = 0.5 * (acc * acc - acc2)
                lr = ci * CH + r
                e1o = lr * F
                tot = tot + e1_v[pl.ds(e1o, 16)]
                tot = tot + m10 * e1_v[pl.ds(e1o + 16, 16)]
                tot = tot + wpad * xd_v[pl.ds(lr * ND, 16)]
                parts = [tot[d] for d in range(D)]
                while len(parts) > 1:
                    parts = [parts[i] + parts[i + 1]
                             for i in range(0, len(parts), 2)]
                svec = jnp.where(lane == r16, parts[0] + b0, svec)
            redsum_v[pl.ds(ci * CH + g * 16, 16)] = svec
            return carry2
        lax.fori_loop(0, CH // 16, groupbody, 0)
        return carry
    lax.fori_loop(0, NCHUNK, chunkbody, 0)

    def gbody(g, carry):
        v = redsum_v[pl.ds(g * 16, 16)]
        o_v[pl.ds(g * 16, 16)] = 1.0 / (1.0 + jnp.exp(-v))
        return carry
    lax.fori_loop(0, RPW // 16, gbody, 0)

    pltpu.sync_copy(o_v, out_h.at[pl.ds(base, RPW)])


@functools.partial(
    pl.kernel,
    mesh=plsc.VectorSubcoreMesh(core_axis_name="c", subcore_axis_name="s"),
    out_type=jax.ShapeDtypeStruct((B,), jnp.float32),
    compiler_params=pltpu.CompilerParams(use_tc_tiling_on_sc=False),
    scratch_types=[
        pltpu.VMEM((NW * ROWS_W // NW, 128), jnp.int32),
        pltpu.VMEM((E1_PAD,), jnp.float32),
        pltpu.VMEM((2, CH * F, D), jnp.float32),
        pltpu.VMEM((XD_PAD,), jnp.float32),
        pltpu.VMEM((16,), jnp.float32),
        pltpu.VMEM((RPW,), jnp.float32),
        pltpu.VMEM((RPW,), jnp.float32),
        pltpu.SemaphoreType.DMA,
        pltpu.SemaphoreType.DMA,
        pltpu.SemaphoreType.DMA,
    ],
)
def _fm_kernel(idx_h, xd_h, wb_h, e1_h, e2_h, out_h, *scratch):
    _fm_body(idx_h, xd_h, wb_h, e1_h, e2_h, out_h, *scratch)


def kernel(X_sparse, X_dense, emb1_tables, emb2_tables, W_dense, b_dense):
    off = jnp.tile(jnp.arange(F, dtype=jnp.int32) * V, B)
    idx = (X_sparse.reshape(B * F) + off).reshape(NW * ROWS_W, 128)
    xd = X_dense.reshape(B * ND)
    wb = jnp.concatenate([
        W_dense[:, 0],
        jnp.zeros((2,), jnp.float32),
        b_dense,
    ])
    e1 = emb1_tables.reshape(F * V)
    e2 = emb2_tables.reshape(F * V, D)
    out = _fm_kernel(idx, xd, wb, e1, e2)
    return out.reshape(B, 1)
